# baseline (speedup 1.0000x reference)
import functools

import jax
import jax.numpy as jnp
from jax.experimental import pallas as pl
from jax.experimental.pallas import tpu as pltpu

B, L, D = 16, 4096, 128
BL = 1024


def _body(x_ref, m_ref, o_ref, acc_ref, cnt_ref):
    li = pl.program_id(1)
    m = m_ref[0, 0, 0]
    s = jnp.dot(m[None, :], x_ref[0],
                preferred_element_type=jnp.float32)
    c = jnp.sum(m)

    @pl.when(li == 0)
    def _init():
        acc_ref[...] = s
        cnt_ref[0] = c

    @pl.when(li > 0)
    def _acc():
        acc_ref[...] += s
        cnt_ref[0] += c

    @pl.when(li == pl.num_programs(1) - 1)
    def _fin():
        o_ref[0] = acc_ref[...] / cnt_ref[0]


@jax.jit
def kernel(inputs, mask):
    m = mask.astype(jnp.float32).reshape(B, L // BL, 1, BL)
    out = pl.pallas_call(
        _body,
        grid=(B, L // BL),
        in_specs=[
            pl.BlockSpec((1, BL, D), lambda b, l: (b, l, 0)),
            pl.BlockSpec((1, 1, 1, BL), lambda b, l: (b, l, 0, 0)),
        ],
        out_specs=pl.BlockSpec((1, 1, D), lambda b, l: (b, 0, 0)),
        out_shape=jax.ShapeDtypeStruct((B, 1, D), jnp.float32),
        scratch_shapes=[
            pltpu.VMEM((1, D), jnp.float32),
            pltpu.SMEM((1,), jnp.float32),
        ],
    )(inputs, m)
    return out.reshape(B, D)


# speedup vs baseline: 2.3520x; 2.3520x over previous
import jax
import jax.numpy as jnp
from jax.experimental import pallas as pl

B, L, D = 16, 4096, 128


def _body(x_ref, m_ref, o_ref):
    m = m_ref[0, 0]
    s = jnp.dot(m[None, :], x_ref[0],
                preferred_element_type=jnp.float32)
    o_ref[0] = s / jnp.sum(m)


@jax.jit
def kernel(inputs, mask):
    m = mask.astype(jnp.float32).reshape(B, 1, L)
    out = pl.pallas_call(
        _body,
        grid=(B,),
        in_specs=[
            pl.BlockSpec((1, L, D), lambda b: (b, 0, 0)),
            pl.BlockSpec((1, 1, L), lambda b: (b, 0, 0)),
        ],
        out_specs=pl.BlockSpec((1, 1, D), lambda b: (b, 0, 0)),
        out_shape=jax.ShapeDtypeStruct((B, 1, D), jnp.float32),
    )(inputs, m)
    return out.reshape(B, D)
